# baseline (speedup 1.0000x reference)
import functools

import jax
import jax.numpy as jnp
from jax import lax
from jax.experimental import pallas as pl
from jax.experimental.pallas import tpu as pltpu
from jax.experimental.pallas import tpu_sc as plsc

_NC = 2
_NS = 16
_NW = _NC * _NS
_CHUNK = 128


@functools.lru_cache(maxsize=None)
def _build_gather(B: int, D: int):
    b_per_w = B // _NW
    n_chunks = b_per_w // _CHUNK
    mesh = plsc.VectorSubcoreMesh(core_axis_name="c", subcore_axis_name="s")

    @functools.partial(
        pl.kernel,
        mesh=mesh,
        out_type=jax.ShapeDtypeStruct((B, D), jnp.float32),
        scratch_types=[
            pltpu.VMEM((n_chunks, _CHUNK), jnp.int32),
            pltpu.VMEM((b_per_w, D), jnp.float32),
            pltpu.SemaphoreType.DMA,
        ],
        compiler_params=pltpu.CompilerParams(use_tc_tiling_on_sc=False),
    )
    def gather(table_hbm, idx_hbm, out_hbm, idx_v, rows_v, sem):
        wid = lax.axis_index("s") * _NC + lax.axis_index("c")
        pltpu.sync_copy(idx_hbm.at[wid], idx_v)
        copies = [
            pltpu.async_copy(
                table_hbm.at[idx_v.at[j]],
                rows_v.at[pl.ds(j * _CHUNK, _CHUNK)],
                sem,
            )
            for j in range(n_chunks)
        ]
        for c in copies:
            c.wait()
        pltpu.sync_copy(rows_v, out_hbm.at[pl.ds(wid * b_per_w, b_per_w)])

    return gather


def kernel(ordered_embs, nodes):
    V, D = ordered_embs.shape
    B = nodes.shape[0]
    b_per_w = B // _NW
    idx = nodes.reshape(_NW, b_per_w // _CHUNK, _CHUNK)
    return _build_gather(B, D)(ordered_embs, idx)


# speedup vs baseline: 1.6786x; 1.6786x over previous
import functools

import jax
import jax.numpy as jnp
from jax import lax
from jax.experimental import pallas as pl
from jax.experimental.pallas import tpu as pltpu
from jax.experimental.pallas import tpu_sc as plsc

_NC = 2
_NS = 16
_NW = _NC * _NS
_K = 16


@functools.lru_cache(maxsize=None)
def _build_gather(B: int, D: int):
    b_per_w = B // _NW
    n_chunks = b_per_w // _K
    mesh = plsc.VectorSubcoreMesh(core_axis_name="c", subcore_axis_name="s")

    @functools.partial(
        pl.kernel,
        mesh=mesh,
        out_type=jax.ShapeDtypeStruct((B, D), jnp.float32),
        scratch_types=[
            pltpu.VMEM((b_per_w,), jnp.int32),
            pltpu.VMEM((b_per_w, D), jnp.float32),
            pltpu.SemaphoreType.DMA,
            pltpu.SemaphoreType.DMA,
        ],
    )
    def gather(table_hbm, idx_hbm, out_hbm, idx_v, rows_v, sem0, sem1):
        wid = lax.axis_index("s") * _NC + lax.axis_index("c")
        pltpu.async_copy(idx_hbm.at[wid], idx_v, sem0).wait()

        def fire(c, sem):
            base = c * _K
            iv = idx_v[pl.ds(base, _K)]
            for j in range(_K):
                pltpu.async_copy(
                    table_hbm.at[iv[j]], rows_v.at[base + j], sem
                )

        def drain(c, sem):
            base = c * _K
            for j in range(_K):
                pltpu.make_async_copy(
                    table_hbm.at[0], rows_v.at[base + j], sem
                ).wait()

        fire(0, sem0)

        def body(i, _):
            c = 2 * i
            fire(c + 1, sem1)
            drain(c, sem0)

            @pl.when(c + 2 < n_chunks)
            def _():
                fire(c + 2, sem0)

            drain(c + 1, sem1)
            return ()

        lax.fori_loop(0, n_chunks // 2, body, (), unroll=False)
        pltpu.sync_copy(rows_v, out_hbm.at[pl.ds(wid * b_per_w, b_per_w)])

    return gather


def kernel(ordered_embs, nodes):
    V, D = ordered_embs.shape
    B = nodes.shape[0]
    idx = nodes.reshape(_NW, B // _NW)
    return _build_gather(B, D)(ordered_embs, idx)
